# baseline (speedup 1.0000x reference)
import jax
import jax.numpy as jnp
from jax import lax
from jax.experimental import pallas as pl
from jax.experimental.pallas import tpu as pltpu
from jax.experimental.pallas import tpu_sc as plsc

_EXTRA = (3216, 3226, 3387, 6617, 6624, 6787,
          2746, 2319, 2445, 2556, 2673,
          6191, 5782, 5905, 6016, 6133)

_B, _V, _J, _E, _C = 4096, 6890, 52, 16, 3
_NC, _NS = 2, 16
_NW = _NC * _NS
_BPW = _B // _NW


def _body(vertices, joints, out, sem):
  wid = lax.axis_index("s") * _NC + lax.axis_index("c")
  base = wid * _BPW
  rows = pl.ds(base, _BPW)
  copies = [pltpu.make_async_copy(
      joints.at[rows], out.at[rows, pl.ds(0, _J)], sem)]
  for j, vidx in enumerate(_EXTRA):
    copies.append(pltpu.make_async_copy(
        vertices.at[rows, vidx], out.at[rows, _J + j], sem))
  for cp in copies:
    cp.start()
  for cp in copies:
    cp.wait()


@jax.jit
def kernel(vertices, joints):
  mesh = plsc.VectorSubcoreMesh(core_axis_name="c", subcore_axis_name="s")
  return pl.kernel(
      _body,
      out_type=jax.ShapeDtypeStruct((_B, _J + _E, _C), jnp.float32),
      mesh=mesh,
      compiler_params=pltpu.CompilerParams(use_tc_tiling_on_sc=False),
      scratch_types=[pltpu.SemaphoreType.DMA],
  )(vertices, joints)


# speedup vs baseline: 27.4849x; 27.4849x over previous
import jax
import jax.numpy as jnp
from jax import lax
from jax.experimental import pallas as pl
from jax.experimental.pallas import tpu as pltpu
from jax.experimental.pallas import tpu_sc as plsc

_EXTRA = (3216, 3226, 3387, 6617, 6624, 6787,
          2746, 2319, 2445, 2556, 2673,
          6191, 5782, 5905, 6016, 6133)

_B, _V, _J, _E, _C = 4096, 6890, 52, 16, 3
_NC, _NS = 2, 16
_NW = _NC * _NS
_BPW = _B // _NW


def _body(vt, jt, out, sem):
  wid = lax.axis_index("s") * _NC + lax.axis_index("c")

  units = [(c, j) for c in range(_C) for j in range(_E)]
  per_worker = {w: [] for w in range(_NW)}
  for u, (c, j) in enumerate(units):
    per_worker[u % _NW].append((c, j))

  for w in range(_NW):
    @pl.when(wid == w)
    def _copies(w=w):
      copies = []
      for c, j in per_worker[w]:
        copies.append(pltpu.make_async_copy(
            vt.at[c, _EXTRA[j]], out.at[c, _J + j], sem))
      if 16 <= w < 16 + _C:
        c = w - 16
        copies.append(pltpu.make_async_copy(
            jt.at[c], out.at[c, pl.ds(0, _J)], sem))
      for cp in copies:
        cp.start()
      for cp in copies:
        cp.wait()


@jax.jit
def kernel(vertices, joints):
  vt = jnp.transpose(vertices, (2, 1, 0))
  jt = jnp.transpose(joints, (2, 1, 0))
  mesh = plsc.VectorSubcoreMesh(core_axis_name="c", subcore_axis_name="s")
  out_t = pl.kernel(
      _body,
      out_type=jax.ShapeDtypeStruct((_C, _J + _E, _B), jnp.float32),
      mesh=mesh,
      compiler_params=pltpu.CompilerParams(use_tc_tiling_on_sc=False),
      scratch_types=[pltpu.SemaphoreType.DMA],
  )(vt, jt)
  return jnp.transpose(out_t, (2, 1, 0))


# speedup vs baseline: 3569.9281x; 129.8869x over previous
import jax
import jax.numpy as jnp
from jax import lax
from jax.experimental import pallas as pl
from jax.experimental.pallas import tpu as pltpu
from jax.experimental.pallas import tpu_sc as plsc

_EXTRA = (3216, 3226, 3387, 6617, 6624, 6787,
          2746, 2319, 2445, 2556, 2673,
          6191, 5782, 5905, 6016, 6133)

_B, _V, _J, _E, _C = 4096, 6890, 52, 16, 3
_NC, _NS = 2, 16
_NW = _NC * _NS
_CB = _B // _NW
_L = 16


def _body(vt, jt, out, gbuf, obuf, sem):
  wid = lax.axis_index("s") * _NC + lax.axis_index("c")
  cols = pl.ds(wid * _CB, _CB)

  for c in range(_C):
    jcopy = pltpu.make_async_copy(
        jt.at[c, :, cols], obuf.at[pl.ds(0, _J), :], sem)
    jcopy.start()
    gcopies = []
    for j, vidx in enumerate(_EXTRA):
      g0 = (vidx // 8) * 8
      gcopies.append(pltpu.make_async_copy(
          vt.at[c, pl.ds(g0, 8), cols], gbuf.at[j], sem))
    for cp in gcopies:
      cp.start()
    jcopy.wait()
    for cp in gcopies:
      cp.wait()

    def extract(k, carry):
      off = pl.multiple_of(k * _L, _L)
      for j, vidx in enumerate(_EXTRA):
        obuf[_J + j, pl.ds(off, _L)] = gbuf[j, vidx % 8, pl.ds(off, _L)]
      return carry

    lax.fori_loop(0, _CB // _L, extract, 0)

    pltpu.sync_copy(obuf, out.at[c, :, cols])


@jax.jit
def kernel(vertices, joints):
  vt = jnp.transpose(vertices, (2, 1, 0))
  jt = jnp.transpose(joints, (2, 1, 0))
  mesh = plsc.VectorSubcoreMesh(core_axis_name="c", subcore_axis_name="s")
  out_t = pl.kernel(
      _body,
      out_type=jax.ShapeDtypeStruct((_C, _J + _E, _B), jnp.float32),
      mesh=mesh,
      scratch_types=[
          pltpu.VMEM((_E, 8, _CB), jnp.float32),
          pltpu.VMEM((_J + _E, _CB), jnp.float32),
          pltpu.SemaphoreType.DMA,
      ],
  )(vt, jt)
  return jnp.transpose(out_t, (2, 1, 0))


# speedup vs baseline: 4015.3431x; 1.1248x over previous
import jax
import jax.numpy as jnp
from jax import lax
from jax.experimental import pallas as pl
from jax.experimental.pallas import tpu as pltpu
from jax.experimental.pallas import tpu_sc as plsc

_EXTRA = (3216, 3226, 3387, 6617, 6624, 6787,
          2746, 2319, 2445, 2556, 2673,
          6191, 5782, 5905, 6016, 6133)

_B, _V, _J, _E, _C = 4096, 6890, 52, 16, 3
_NC, _NS = 2, 16
_NW = _NC * _NS
_CB = _B // _NW
_L = 16


def _body(vt, jt, out, gbuf, obuf, sem):
  wid = lax.axis_index("s") * _NC + lax.axis_index("c")
  cols = pl.ds(wid * _CB, _CB)

  copies = [pltpu.make_async_copy(
      jt.at[:, :, cols], obuf.at[:, pl.ds(0, _J), :], sem)]
  for j, vidx in enumerate(_EXTRA):
    g0 = (vidx // 8) * 8
    copies.append(pltpu.make_async_copy(
        vt.at[:, pl.ds(g0, 8), cols], gbuf.at[j], sem))
  for cp in copies:
    cp.start()
  for cp in copies:
    cp.wait()

  def extract(k, carry):
    off = pl.multiple_of(k * _L, _L)
    for j, vidx in enumerate(_EXTRA):
      for c in range(_C):
        obuf[c, _J + j, pl.ds(off, _L)] = gbuf[j, c, vidx % 8, pl.ds(off, _L)]
    return carry

  lax.fori_loop(0, _CB // _L, extract, 0)

  pltpu.sync_copy(obuf, out.at[:, :, cols])


@jax.jit
def kernel(vertices, joints):
  vt = jnp.transpose(vertices, (2, 1, 0))
  jt = jnp.transpose(joints, (2, 1, 0))
  mesh = plsc.VectorSubcoreMesh(core_axis_name="c", subcore_axis_name="s")
  out_t = pl.kernel(
      _body,
      out_type=jax.ShapeDtypeStruct((_C, _J + _E, _B), jnp.float32),
      mesh=mesh,
      scratch_types=[
          pltpu.VMEM((_E, _C, 8, _CB), jnp.float32),
          pltpu.VMEM((_C, _J + _E, _CB), jnp.float32),
          pltpu.SemaphoreType.DMA,
      ],
  )(vt, jt)
  return jnp.transpose(out_t, (2, 1, 0))
